# baseline (speedup 1.0000x reference)
import functools

import jax
import jax.numpy as jnp
from jax import lax
from jax.experimental import pallas as pl
from jax.experimental.pallas import tpu as pltpu
from jax.experimental.pallas import tpu_sc as plsc

T, N, E, D, H, G, A, S = 8, 10000, 320000, 128, 128, 64, 32, 8

NW = 32
EPW = E // NW
CHUNK = 80
NCHUNK = EPW // CHUNK
NPAD = 10240
RPT = NPAD // 16
ZROWS = 128
DEGW = 16

_MESH = plsc.VectorSubcoreMesh(core_axis_name="c", subcore_axis_name="s",
                               num_cores=2, num_subcores=16)


def _zero_fill_2d(ref, rows, width):
    zv = jnp.zeros((16,), jnp.float32)

    def body(r, carry):
        for cc in range(width // 16):
            ref[r, pl.ds(cc * 16, 16)] = zv
        return carry

    lax.fori_loop(0, rows, body, 0)


def _make_agg(nt):
    @functools.partial(
        pl.kernel,
        mesh=_MESH,
        out_type=jax.ShapeDtypeStruct((nt * 2 * NPAD, H), jnp.float32),
        scratch_types=[
            pltpu.VMEM((3, CHUNK), jnp.int32),
            pltpu.VMEM((3, CHUNK), jnp.int32),
            pltpu.VMEM((3, CHUNK, H), jnp.float32),
            pltpu.VMEM_SHARED((NPAD, H), jnp.float32),
            pltpu.SemaphoreType.DMA,
            pltpu.SemaphoreType.DMA,
            pltpu.SemaphoreType.DMA,
            pltpu.SemaphoreType.DMA,
            pltpu.SemaphoreType.DMA,
            pltpu.SemaphoreType.DMA,
        ],
    )
    def agg(hp_hbm, src_hbm, dst_hbm, out,
            src_v, dst_v, rows_v, acc,
            sem_i0, sem_i1, sem_i2, sem_g0, sem_g1, sem_g2):
        core = lax.axis_index("c")
        sub = lax.axis_index("s")
        wid = core * 16 + sub
        sem_i = (sem_i0, sem_i1, sem_i2)
        sem_g = (sem_g0, sem_g1, sem_g2)

        def issue_idx(c, s):
            pltpu.async_copy(src_hbm.at[wid, c], src_v.at[s], sem_i[s])
            pltpu.async_copy(dst_hbm.at[wid, c], dst_v.at[s], sem_i[s])

        def wait_idx(s, toff):
            pltpu.make_async_copy(src_hbm.at[wid, 0], src_v.at[s], sem_i[s]).wait()
            pltpu.make_async_copy(dst_hbm.at[wid, 0], dst_v.at[s], sem_i[s]).wait()
            if toff:
                for j in range(CHUNK // 16):
                    sl = pl.ds(j * 16, 16)
                    src_v[s, sl] = src_v[s, sl] + toff

        def gath(s):
            pltpu.async_copy(hp_hbm.at[src_v.at[s]], rows_v.at[s], sem_g[s])

        def wgath(s):
            pltpu.make_async_copy(hp_hbm.at[src_v.at[s]], rows_v.at[s],
                                  sem_g[s]).wait()

        def scat(s):
            pltpu.sync_copy(rows_v.at[s], acc.at[dst_v.at[s]], add=True)

        for t in range(nt):
            toff = t * NPAD

            _zero_fill_2d(rows_v.at[0], CHUNK, H)
            for z in range(RPT // CHUNK):
                pltpu.sync_copy(rows_v.at[0],
                                acc.at[pl.ds(sub * RPT + z * CHUNK, CHUNK)])
            plsc.subcore_barrier()

            for s in range(3):
                issue_idx(s, s)
            wait_idx(0, toff)
            gath(0)
            wait_idx(1, toff)
            gath(1)

            def step(c, s):
                wait_idx((s + 2) % 3, toff)
                gath((s + 2) % 3)
                wgath(s)
                scat(s)
                issue_idx(c + 3, s)

            def tri_body(k, carry):
                c0 = 3 * k
                step(c0, 0)
                step(c0 + 1, 1)
                step(c0 + 2, 2)
                return carry

            lax.fori_loop(0, (NCHUNK - 2) // 3, tri_body, 0)
            wgath(0)
            scat(0)
            wgath(1)
            scat(1)
            pltpu.make_async_copy(src_hbm.at[wid, 0], src_v.at[2],
                                  sem_i[2]).wait()
            pltpu.make_async_copy(dst_hbm.at[wid, 0], dst_v.at[2],
                                  sem_i[2]).wait()
            plsc.subcore_barrier()

            orow = pl.multiple_of(t * 2 * NPAD + core * NPAD + sub * RPT, 8)
            pltpu.sync_copy(acc.at[pl.ds(sub * RPT, RPT)],
                            out.at[pl.ds(orow, RPT)])
            plsc.subcore_barrier()

    return agg


_agg_kernel = _make_agg(1)
_agg_kernel_t = _make_agg(T)


BN = 1024
NBLK = NPAD // BN


def _dis_kernel_body(d0_ref, d1_ref, out_ref):
    deg = d0_ref[:, 0:1] + d1_ref[:, 0:1] + 1.0
    out_ref[...] = jnp.broadcast_to(lax.rsqrt(deg), (BN, DEGW))


def _dis_kernel(degp):
    return pl.pallas_call(
        _dis_kernel_body,
        grid=(NBLK,),
        in_specs=[
            pl.BlockSpec((BN, H), lambda nb: (nb, 0)),
            pl.BlockSpec((BN, H), lambda nb: (nb + NBLK, 0)),
        ],
        out_specs=pl.BlockSpec((BN, DEGW), lambda nb: (nb, 0)),
        out_shape=jax.ShapeDtypeStruct((NPAD, DEGW), jnp.float32),
    )(degp, degp)


def _first_layer_body(x_ref, w_ref, dis_ref, out_ref):
    dis = dis_ref[:, 0:1]
    h = jnp.dot(x_ref[0], w_ref[...], preferred_element_type=jnp.float32)
    out_ref[...] = h * dis


def _first_layer(x_seq, W0, dis16):
    return pl.pallas_call(
        _first_layer_body,
        grid=(T, NBLK),
        in_specs=[
            pl.BlockSpec((1, BN, D), lambda t, nb: (t, nb, 0)),
            pl.BlockSpec((D, H), lambda t, nb: (0, 0)),
            pl.BlockSpec((BN, DEGW), lambda t, nb: (nb, 0)),
        ],
        out_specs=pl.BlockSpec((BN, H), lambda t, nb: (t * NBLK + nb, 0)),
        out_shape=jax.ShapeDtypeStruct((T * NPAD, H), jnp.float32),
    )(x_seq, W0, dis16)


def _mid_layer_body(p0_ref, p1_ref, hp_ref, dis_ref, b_ref, w_ref, out_ref):
    dis = dis_ref[:, 0:1]
    u = dis * (p0_ref[...] + p1_ref[...] + hp_ref[...]) + b_ref[...]
    u = jnp.maximum(u, 0.0)
    out_ref[...] = jnp.dot(u, w_ref[...], preferred_element_type=jnp.float32) * dis


def _mid_layer(P, hp, dis16, b, W_next):
    return pl.pallas_call(
        _mid_layer_body,
        grid=(T, NBLK),
        in_specs=[
            pl.BlockSpec((BN, H), lambda t, nb: (t * 2 * NBLK + nb, 0)),
            pl.BlockSpec((BN, H), lambda t, nb: (t * 2 * NBLK + NBLK + nb, 0)),
            pl.BlockSpec((BN, H), lambda t, nb: (t * NBLK + nb, 0)),
            pl.BlockSpec((BN, DEGW), lambda t, nb: (nb, 0)),
            pl.BlockSpec((1, H), lambda t, nb: (0, 0)),
            pl.BlockSpec((H, H), lambda t, nb: (0, 0)),
        ],
        out_specs=pl.BlockSpec((BN, H), lambda t, nb: (t * NBLK + nb, 0)),
        out_shape=jax.ShapeDtypeStruct((T * NPAD, H), jnp.float32),
    )(P, P, hp, dis16, b, W_next)


def _last_layer_body(p0_ref, p1_ref, hp_ref, dis_ref, b_ref, bid_ref,
                     pooled_ref, counts_ref):
    t = pl.program_id(0)
    nb = pl.program_id(1)
    dis = dis_ref[:, 0:1]
    u = dis * (p0_ref[...] + p1_ref[...] + hp_ref[...]) + b_ref[...]
    u = jnp.maximum(u, 0.0)
    rowid = nb * BN + lax.broadcasted_iota(jnp.int32, (BN, 1), 0)
    u = jnp.where(rowid < N, u, 0.0)
    bid = bid_ref[0, 0, :]
    iota_g = lax.broadcasted_iota(jnp.int32, (BN, G), 1)
    onehot = (bid[:, None] == iota_g).astype(jnp.float32)
    pooled_b = lax.dot_general(onehot, u, (((0,), (0,)), ((), ())),
                               preferred_element_type=jnp.float32)

    @pl.when(nb == 0)
    def _():
        pooled_ref[0] = pooled_b

    @pl.when(nb > 0)
    def _():
        pooled_ref[0] += pooled_b

    @pl.when(t == 0)
    def _():
        counts_b = jnp.sum(onehot, axis=0, keepdims=True)

        @pl.when(nb == 0)
        def _():
            counts_ref[...] = counts_b

        @pl.when(nb > 0)
        def _():
            counts_ref[...] += counts_b


def _last_layer(P, hp, dis16, b, bids3d):
    return pl.pallas_call(
        _last_layer_body,
        grid=(T, NBLK),
        in_specs=[
            pl.BlockSpec((BN, H), lambda t, nb: (t * 2 * NBLK + nb, 0)),
            pl.BlockSpec((BN, H), lambda t, nb: (t * 2 * NBLK + NBLK + nb, 0)),
            pl.BlockSpec((BN, H), lambda t, nb: (t * NBLK + nb, 0)),
            pl.BlockSpec((BN, DEGW), lambda t, nb: (nb, 0)),
            pl.BlockSpec((1, H), lambda t, nb: (0, 0)),
            pl.BlockSpec((1, 1, BN), lambda t, nb: (nb, 0, 0)),
        ],
        out_specs=[
            pl.BlockSpec((1, G, H), lambda t, nb: (t, 0, 0)),
            pl.BlockSpec((1, G), lambda t, nb: (0, 0)),
        ],
        out_shape=[
            jax.ShapeDtypeStruct((T, G, H), jnp.float32),
            jax.ShapeDtypeStruct((1, G), jnp.float32),
        ],
    )(P, P, hp, dis16, b, bids3d)


def _sigmoid(x):
    return 1.0 / (1.0 + jnp.exp(-x))


def _gru_heads_body(pooled_ref, counts_ref, wih_ref, whh_ref, bih_ref,
                    bhh_ref, wc_ref, bc_ref, ws_ref, bs_ref,
                    coord_ref, ss_ref):
    inv = 1.0 / jnp.maximum(counts_ref[0, :], 1.0)
    h = jnp.zeros((G, H), jnp.float32)
    for t in range(T):
        xt = pooled_ref[t] * inv[:, None]
        gi = lax.dot_general(xt, wih_ref[...], (((1,), (1,)), ((), ())),
                             preferred_element_type=jnp.float32) + bih_ref[...]
        gh = lax.dot_general(h, whh_ref[...], (((1,), (1,)), ((), ())),
                             preferred_element_type=jnp.float32) + bhh_ref[...]
        r = _sigmoid(gi[:, 0:H] + gh[:, 0:H])
        z = _sigmoid(gi[:, H:2 * H] + gh[:, H:2 * H])
        n = jnp.tanh(gi[:, 2 * H:] + r * gh[:, 2 * H:])
        h = (1.0 - z) * n + z * h
        coord_ref[t] = jnp.dot(h, wc_ref[...],
                               preferred_element_type=jnp.float32) + bc_ref[...]
        ss_ref[t] = jnp.dot(h, ws_ref[...],
                            preferred_element_type=jnp.float32) + bs_ref[...]


def _gru_heads(pooled, counts, W_ih, W_hh, b_ih, b_hh,
               W_coord, b_coord, W_ss, b_ss):
    return pl.pallas_call(
        _gru_heads_body,
        out_shape=[
            jax.ShapeDtypeStruct((T, G, A * 3), jnp.float32),
            jax.ShapeDtypeStruct((T, G, A * S), jnp.float32),
        ],
    )(pooled, counts, W_ih, W_hh, b_ih, b_hh,
      W_coord, b_coord, W_ss, b_ss)


def kernel(x_seq, edge_index, batch_ids, W0, b0, W1, b1, W2, b2, W3, b3,
           W_ih, W_hh, b_ih, b_hh, W_coord, b_coord, W_ss, b_ss):
    pad = jnp.zeros((NW, 2, CHUNK), jnp.int32)
    src2d = jnp.concatenate([edge_index[0].reshape(NW, NCHUNK, CHUNK), pad], 1)
    dst2d = jnp.concatenate([edge_index[1].reshape(NW, NCHUNK, CHUNK), pad], 1)
    bids_pad = jnp.full((NPAD,), G, jnp.int32).at[:N].set(batch_ids)
    bids3d = bids_pad.reshape(NBLK, 1, BN)
    b0r, b1r, b2r, b3r = (b.reshape(1, H) for b in (b0, b1, b2, b3))

    ones_mat = jnp.ones((NPAD, H), jnp.float32)
    degp = _agg_kernel(ones_mat, src2d, dst2d)
    dis16 = _dis_kernel(degp)

    hp = _first_layer(x_seq, W0, dis16)

    for (b, W_next) in [(b0r, W1), (b1r, W2), (b2r, W3)]:
        P = _agg_kernel_t(hp, src2d, dst2d)
        hp = _mid_layer(P, hp, dis16, b, W_next)

    P = _agg_kernel_t(hp, src2d, dst2d)
    pooled, counts = _last_layer(P, hp, dis16, b3r, bids3d)

    coord_r, ss_r = _gru_heads(
        pooled, counts, W_ih, W_hh, b_ih.reshape(1, 3 * H),
        b_hh.reshape(1, 3 * H), W_coord, b_coord.reshape(1, A * 3),
        W_ss, b_ss.reshape(1, A * S))

    coord = jnp.transpose(coord_r, (1, 0, 2)).reshape(G, T, A, 3)
    ss = jnp.transpose(ss_r, (1, 0, 2)).reshape(G, T, A, S)
    return (coord, ss)


# speedup vs baseline: 1.0394x; 1.0394x over previous
import functools

import jax
import jax.numpy as jnp
from jax import lax
from jax.experimental import pallas as pl
from jax.experimental.pallas import tpu as pltpu
from jax.experimental.pallas import tpu_sc as plsc

T, N, E, D, H, G, A, S = 8, 10000, 320000, 128, 128, 64, 32, 8

NW = 32
EPW = E // NW
CHUNK = 80
NCHUNK = EPW // CHUNK
NPAD = 10240
RPT = NPAD // 16
ZROWS = 128
DEGW = 16

_MESH = plsc.VectorSubcoreMesh(core_axis_name="c", subcore_axis_name="s",
                               num_cores=2, num_subcores=16)


def _zero_fill_2d(ref, rows, width):
    zv = jnp.zeros((16,), jnp.float32)

    def body(r, carry):
        for cc in range(width // 16):
            ref[r, pl.ds(cc * 16, 16)] = zv
        return carry

    lax.fori_loop(0, rows, body, 0)


def _make_agg(nt):
    @functools.partial(
        pl.kernel,
        mesh=_MESH,
        out_type=jax.ShapeDtypeStruct((nt * 2 * NPAD, H), jnp.float32),
        scratch_types=[
            pltpu.VMEM((3, CHUNK), jnp.int32),
            pltpu.VMEM((3, CHUNK), jnp.int32),
            pltpu.VMEM((3, CHUNK, H), jnp.float32),
            pltpu.VMEM_SHARED((NPAD, H), jnp.float32),
            pltpu.SemaphoreType.DMA,
            pltpu.SemaphoreType.DMA,
            pltpu.SemaphoreType.DMA,
            pltpu.SemaphoreType.DMA,
            pltpu.SemaphoreType.DMA,
            pltpu.SemaphoreType.DMA,
        ],
    )
    def agg(hp_hbm, src_hbm, dst_hbm, out,
            src_v, dst_v, rows_v, acc,
            sem_i0, sem_i1, sem_i2, sem_g0, sem_g1, sem_g2):
        core = lax.axis_index("c")
        sub = lax.axis_index("s")
        wid = core * 16 + sub
        sem_i = (sem_i0, sem_i1, sem_i2)
        sem_g = (sem_g0, sem_g1, sem_g2)

        def issue_idx(c, s):
            pltpu.async_copy(src_hbm.at[wid, c], src_v.at[s], sem_i[s])
            pltpu.async_copy(dst_hbm.at[wid, c], dst_v.at[s], sem_i[s])

        def wait_idx(s, toff):
            pltpu.make_async_copy(src_hbm.at[wid, 0], src_v.at[s], sem_i[s]).wait()
            pltpu.make_async_copy(dst_hbm.at[wid, 0], dst_v.at[s], sem_i[s]).wait()
            if toff:
                for j in range(CHUNK // 16):
                    sl = pl.ds(j * 16, 16)
                    src_v[s, sl] = src_v[s, sl] + toff

        def gath(s):
            pltpu.async_copy(hp_hbm.at[src_v.at[s]], rows_v.at[s], sem_g[s])

        def wgath(s):
            pltpu.make_async_copy(hp_hbm.at[src_v.at[s]], rows_v.at[s],
                                  sem_g[s]).wait()

        def scat(s):
            pltpu.sync_copy(rows_v.at[s], acc.at[dst_v.at[s]], add=True)

        for t in range(nt):
            toff = t * NPAD

            _zero_fill_2d(rows_v.at[0], CHUNK, H)
            for z in range(RPT // CHUNK):
                pltpu.sync_copy(rows_v.at[0],
                                acc.at[pl.ds(sub * RPT + z * CHUNK, CHUNK)])
            plsc.subcore_barrier()

            for s in range(3):
                issue_idx(s, s)
            wait_idx(0, toff)
            gath(0)
            wait_idx(1, toff)
            gath(1)

            def step(c, s):
                wait_idx((s + 2) % 3, toff)
                gath((s + 2) % 3)
                wgath(s)
                scat(s)
                issue_idx(c + 3, s)

            def tri_body(k, carry):
                c0 = 3 * k
                step(c0, 0)
                step(c0 + 1, 1)
                step(c0 + 2, 2)
                return carry

            lax.fori_loop(0, (NCHUNK - 2) // 3, tri_body, 0)
            wgath(0)
            scat(0)
            wgath(1)
            scat(1)
            pltpu.make_async_copy(src_hbm.at[wid, 0], src_v.at[2],
                                  sem_i[2]).wait()
            pltpu.make_async_copy(dst_hbm.at[wid, 0], dst_v.at[2],
                                  sem_i[2]).wait()
            plsc.subcore_barrier()

            orow = pl.multiple_of(t * 2 * NPAD + core * NPAD + sub * RPT, 8)
            pltpu.sync_copy(acc.at[pl.ds(sub * RPT, RPT)],
                            out.at[pl.ds(orow, RPT)])
            plsc.subcore_barrier()

    return agg


_agg_kernel = _make_agg(1)
_agg_kernel_t = _make_agg(T)


BN = 1024
NBLK = NPAD // BN


def _dis_kernel_body(d0_ref, d1_ref, out_ref):
    deg = d0_ref[:, 0:1] + d1_ref[:, 0:1] + 1.0
    out_ref[...] = jnp.broadcast_to(lax.rsqrt(deg), (BN, DEGW))


def _dis_kernel(degp):
    return pl.pallas_call(
        _dis_kernel_body,
        grid=(NBLK,),
        in_specs=[
            pl.BlockSpec((BN, H), lambda nb: (nb, 0)),
            pl.BlockSpec((BN, H), lambda nb: (nb + NBLK, 0)),
        ],
        out_specs=pl.BlockSpec((BN, DEGW), lambda nb: (nb, 0)),
        out_shape=jax.ShapeDtypeStruct((NPAD, DEGW), jnp.float32),
    )(degp, degp)


def _first_layer_body(x_ref, w_ref, dis_ref, out_ref):
    dis = dis_ref[:, 0:1]
    h = jnp.dot(x_ref[0], w_ref[...], preferred_element_type=jnp.float32)
    out_ref[0] = h * dis


def _first_layer(x_seq, W0, dis16):
    return pl.pallas_call(
        _first_layer_body,
        grid=(T, NBLK),
        in_specs=[
            pl.BlockSpec((1, BN, D), lambda t, nb: (t, nb, 0)),
            pl.BlockSpec((D, H), lambda t, nb: (0, 0)),
            pl.BlockSpec((BN, DEGW), lambda t, nb: (nb, 0)),
        ],
        out_specs=pl.BlockSpec((1, BN, H), lambda t, nb: (t, nb, 0)),
        out_shape=jax.ShapeDtypeStruct((T, NPAD, H), jnp.float32),
    )(x_seq, W0, dis16)


def _mid_layer_body(p0_ref, p1_ref, hp_ref, dis_ref, b_ref, w_ref, out_ref):
    dis = dis_ref[:, 0:1]
    u = dis * (p0_ref[...] + p1_ref[...] + hp_ref[...]) + b_ref[...]
    u = jnp.maximum(u, 0.0)
    out_ref[...] = jnp.dot(u, w_ref[...], preferred_element_type=jnp.float32) * dis


def _mid_layer(P, hp, dis16, b, W_next):
    return pl.pallas_call(
        _mid_layer_body,
        grid=(NBLK,),
        in_specs=[
            pl.BlockSpec((BN, H), lambda nb: (nb, 0)),
            pl.BlockSpec((BN, H), lambda nb: (nb + NBLK, 0)),
            pl.BlockSpec((BN, H), lambda nb: (nb, 0)),
            pl.BlockSpec((BN, DEGW), lambda nb: (nb, 0)),
            pl.BlockSpec((1, H), lambda nb: (0, 0)),
            pl.BlockSpec((H, H), lambda nb: (0, 0)),
        ],
        out_specs=pl.BlockSpec((BN, H), lambda nb: (nb, 0)),
        out_shape=jax.ShapeDtypeStruct((NPAD, H), jnp.float32),
    )(P, P, hp, dis16, b, W_next)


def _last_layer_body(p0_ref, p1_ref, hp_ref, dis_ref, b_ref, bid_ref,
                     pooled_ref, counts_ref):
    nb = pl.program_id(0)
    dis = dis_ref[:, 0:1]
    u = dis * (p0_ref[...] + p1_ref[...] + hp_ref[...]) + b_ref[...]
    u = jnp.maximum(u, 0.0)
    rowid = nb * BN + lax.broadcasted_iota(jnp.int32, (BN, 1), 0)
    u = jnp.where(rowid < N, u, 0.0)
    bid = bid_ref[0, 0, :]
    iota_g = lax.broadcasted_iota(jnp.int32, (BN, G), 1)
    onehot = (bid[:, None] == iota_g).astype(jnp.float32)
    pooled_b = lax.dot_general(onehot, u, (((0,), (0,)), ((), ())),
                               preferred_element_type=jnp.float32)
    counts_b = jnp.sum(onehot, axis=0, keepdims=True)

    @pl.when(nb == 0)
    def _():
        pooled_ref[...] = pooled_b
        counts_ref[...] = counts_b

    @pl.when(nb > 0)
    def _():
        pooled_ref[...] += pooled_b
        counts_ref[...] += counts_b


def _last_layer(P, hp, dis16, b, bids3d):
    return pl.pallas_call(
        _last_layer_body,
        grid=(NBLK,),
        in_specs=[
            pl.BlockSpec((BN, H), lambda nb: (nb, 0)),
            pl.BlockSpec((BN, H), lambda nb: (nb + NBLK, 0)),
            pl.BlockSpec((BN, H), lambda nb: (nb, 0)),
            pl.BlockSpec((BN, DEGW), lambda nb: (nb, 0)),
            pl.BlockSpec((1, H), lambda nb: (0, 0)),
            pl.BlockSpec((1, 1, BN), lambda nb: (nb, 0, 0)),
        ],
        out_specs=[
            pl.BlockSpec((G, H), lambda nb: (0, 0)),
            pl.BlockSpec((1, G), lambda nb: (0, 0)),
        ],
        out_shape=[
            jax.ShapeDtypeStruct((G, H), jnp.float32),
            jax.ShapeDtypeStruct((1, G), jnp.float32),
        ],
    )(P, P, hp, dis16, b, bids3d)


def _sigmoid(x):
    return 1.0 / (1.0 + jnp.exp(-x))


def _gru_heads_body(*refs):
    pooled_refs = refs[:T]
    (counts_ref, wih_ref, whh_ref, bih_ref, bhh_ref,
     wc_ref, bc_ref, ws_ref, bs_ref, coord_ref, ss_ref) = refs[T:]
    inv = 1.0 / jnp.maximum(counts_ref[0, :], 1.0)
    h = jnp.zeros((G, H), jnp.float32)
    for t in range(T):
        xt = pooled_refs[t][...] * inv[:, None]
        gi = lax.dot_general(xt, wih_ref[...], (((1,), (1,)), ((), ())),
                             preferred_element_type=jnp.float32) + bih_ref[...]
        gh = lax.dot_general(h, whh_ref[...], (((1,), (1,)), ((), ())),
                             preferred_element_type=jnp.float32) + bhh_ref[...]
        r = _sigmoid(gi[:, 0:H] + gh[:, 0:H])
        z = _sigmoid(gi[:, H:2 * H] + gh[:, H:2 * H])
        n = jnp.tanh(gi[:, 2 * H:] + r * gh[:, 2 * H:])
        h = (1.0 - z) * n + z * h
        coord_ref[t] = jnp.dot(h, wc_ref[...],
                               preferred_element_type=jnp.float32) + bc_ref[...]
        ss_ref[t] = jnp.dot(h, ws_ref[...],
                            preferred_element_type=jnp.float32) + bs_ref[...]


def _gru_heads(pooled_list, counts, W_ih, W_hh, b_ih, b_hh,
               W_coord, b_coord, W_ss, b_ss):
    return pl.pallas_call(
        _gru_heads_body,
        out_shape=[
            jax.ShapeDtypeStruct((T, G, A * 3), jnp.float32),
            jax.ShapeDtypeStruct((T, G, A * S), jnp.float32),
        ],
    )(*pooled_list, counts, W_ih, W_hh, b_ih, b_hh,
      W_coord, b_coord, W_ss, b_ss)


def kernel(x_seq, edge_index, batch_ids, W0, b0, W1, b1, W2, b2, W3, b3,
           W_ih, W_hh, b_ih, b_hh, W_coord, b_coord, W_ss, b_ss):
    pad = jnp.zeros((NW, 2, CHUNK), jnp.int32)
    src2d = jnp.concatenate([edge_index[0].reshape(NW, NCHUNK, CHUNK), pad], 1)
    dst2d = jnp.concatenate([edge_index[1].reshape(NW, NCHUNK, CHUNK), pad], 1)
    bids_pad = jnp.full((NPAD,), G, jnp.int32).at[:N].set(batch_ids)
    bids3d = bids_pad.reshape(NBLK, 1, BN)
    b0r, b1r, b2r, b3r = (b.reshape(1, H) for b in (b0, b1, b2, b3))

    ones_mat = jnp.ones((NPAD, H), jnp.float32)
    degp = _agg_kernel(ones_mat, src2d, dst2d)
    dis16 = _dis_kernel(degp)

    hp0 = _first_layer(x_seq, W0, dis16)
    hp = [hp0[t] for t in range(T)]

    for (b, W_next) in [(b0r, W1), (b1r, W2), (b2r, W3)]:
        nxt = []
        for t in range(T):
            P = _agg_kernel(hp[t], src2d, dst2d)
            nxt.append(_mid_layer(P, hp[t], dis16, b, W_next))
        hp = nxt

    pooled_list = []
    counts = None
    for t in range(T):
        P = _agg_kernel(hp[t], src2d, dst2d)
        pooled_t, counts_t = _last_layer(P, hp[t], dis16, b3r, bids3d)
        pooled_list.append(pooled_t)
        if counts is None:
            counts = counts_t

    coord_r, ss_r = _gru_heads(
        pooled_list, counts, W_ih, W_hh, b_ih.reshape(1, 3 * H),
        b_hh.reshape(1, 3 * H), W_coord, b_coord.reshape(1, A * 3),
        W_ss, b_ss.reshape(1, A * S))

    coord = jnp.transpose(coord_r, (1, 0, 2)).reshape(G, T, A, 3)
    ss = jnp.transpose(ss_r, (1, 0, 2)).reshape(G, T, A, S)
    return (coord, ss)


# speedup vs baseline: 1.0474x; 1.0077x over previous
import functools

import jax
import jax.numpy as jnp
from jax import lax
from jax.experimental import pallas as pl
from jax.experimental.pallas import tpu as pltpu
from jax.experimental.pallas import tpu_sc as plsc

T, N, E, D, H, G, A, S = 8, 10000, 320000, 128, 128, 64, 32, 8

NW = 32
EPW = E // NW
CHUNK = 80
NCHUNK = EPW // CHUNK
NPAD = 10240
RPT = NPAD // 16
DEGW = 16

_MESH = plsc.VectorSubcoreMesh(core_axis_name="c", subcore_axis_name="s",
                               num_cores=2, num_subcores=16)


def _zero_fill_2d(ref, rows, width):
    zv = jnp.zeros((16,), jnp.float32)

    def body(r, carry):
        for cc in range(width // 16):
            ref[r, pl.ds(cc * 16, 16)] = zv
        return carry

    lax.fori_loop(0, rows, body, 0)


def _make_agg(nt):
    @functools.partial(
        pl.kernel,
        mesh=_MESH,
        out_type=jax.ShapeDtypeStruct((nt * 2 * NPAD, H), jnp.float32),
        scratch_types=[
            pltpu.VMEM((3, CHUNK), jnp.int32),
            pltpu.VMEM((3, CHUNK), jnp.int32),
            pltpu.VMEM((3, CHUNK, H), jnp.float32),
            pltpu.VMEM_SHARED((NPAD, H), jnp.float32),
            pltpu.SemaphoreType.DMA,
            pltpu.SemaphoreType.DMA,
            pltpu.SemaphoreType.DMA,
            pltpu.SemaphoreType.DMA,
            pltpu.SemaphoreType.DMA,
            pltpu.SemaphoreType.DMA,
        ],
    )
    def agg(hp_hbm, src_hbm, dst_hbm, out,
            src_v, dst_v, rows_v, acc,
            sem_i0, sem_i1, sem_i2, sem_g0, sem_g1, sem_g2):
        core = lax.axis_index("c")
        sub = lax.axis_index("s")
        wid = core * 16 + sub
        sem_i = (sem_i0, sem_i1, sem_i2)
        sem_g = (sem_g0, sem_g1, sem_g2)

        def issue_idx(c, s):
            pltpu.async_copy(src_hbm.at[wid, c], src_v.at[s], sem_i[s])
            pltpu.async_copy(dst_hbm.at[wid, c], dst_v.at[s], sem_i[s])

        def wait_idx(s, toff):
            pltpu.make_async_copy(src_hbm.at[wid, 0], src_v.at[s], sem_i[s]).wait()
            pltpu.make_async_copy(dst_hbm.at[wid, 0], dst_v.at[s], sem_i[s]).wait()
            if toff:
                for j in range(CHUNK // 16):
                    sl = pl.ds(j * 16, 16)
                    src_v[s, sl] = src_v[s, sl] + toff

        def gath(s):
            pltpu.async_copy(hp_hbm.at[src_v.at[s]], rows_v.at[s], sem_g[s])

        def wgath(s):
            pltpu.make_async_copy(hp_hbm.at[src_v.at[s]], rows_v.at[s],
                                  sem_g[s]).wait()

        def scat(s):
            pltpu.sync_copy(rows_v.at[s], acc.at[dst_v.at[s]], add=True)

        for t in range(nt):
            toff = t * NPAD

            _zero_fill_2d(rows_v.at[0], CHUNK, H)
            for z in range(RPT // CHUNK):
                pltpu.sync_copy(rows_v.at[0],
                                acc.at[pl.ds(sub * RPT + z * CHUNK, CHUNK)])
            plsc.subcore_barrier()

            for s in range(3):
                issue_idx(s, s)
            wait_idx(0, toff)
            gath(0)
            wait_idx(1, toff)
            gath(1)

            def step(c, s):
                wait_idx((s + 2) % 3, toff)
                gath((s + 2) % 3)
                wgath(s)
                scat(s)
                issue_idx(c + 3, s)

            def tri_body(k, carry):
                c0 = 3 * k
                step(c0, 0)
                step(c0 + 1, 1)
                step(c0 + 2, 2)
                return carry

            lax.fori_loop(0, (NCHUNK - 2) // 3, tri_body, 0)
            wgath(0)
            scat(0)
            wgath(1)
            scat(1)
            pltpu.make_async_copy(src_hbm.at[wid, 0], src_v.at[2],
                                  sem_i[2]).wait()
            pltpu.make_async_copy(dst_hbm.at[wid, 0], dst_v.at[2],
                                  sem_i[2]).wait()
            plsc.subcore_barrier()

            orow = pl.multiple_of(t * 2 * NPAD + core * NPAD + sub * RPT, 8)
            pltpu.sync_copy(acc.at[pl.ds(sub * RPT, RPT)],
                            out.at[pl.ds(orow, RPT)])
            plsc.subcore_barrier()

    return agg


_agg_kernel = _make_agg(1)


BN = 1024
NBLK = NPAD // BN


def _dis_kernel_body(d0_ref, d1_ref, out_ref):
    deg = d0_ref[:, 0:1] + d1_ref[:, 0:1] + 1.0
    out_ref[...] = jnp.broadcast_to(lax.rsqrt(deg), (BN, DEGW))


def _dis_kernel(degp):
    return pl.pallas_call(
        _dis_kernel_body,
        grid=(NBLK,),
        in_specs=[
            pl.BlockSpec((BN, H), lambda nb: (nb, 0)),
            pl.BlockSpec((BN, H), lambda nb: (nb + NBLK, 0)),
        ],
        out_specs=pl.BlockSpec((BN, DEGW), lambda nb: (nb, 0)),
        out_shape=jax.ShapeDtypeStruct((NPAD, DEGW), jnp.float32),
    )(degp, degp)


def _first_layer_body(x_ref, w_ref, dis_ref, out_ref):
    dis = dis_ref[:, 0:1]
    h = jnp.dot(x_ref[0], w_ref[...], preferred_element_type=jnp.float32)
    out_ref[0] = h * dis


def _first_layer(x_seq, W0, dis16):
    return pl.pallas_call(
        _first_layer_body,
        grid=(T, NBLK),
        in_specs=[
            pl.BlockSpec((1, BN, D), lambda t, nb: (t, nb, 0)),
            pl.BlockSpec((D, H), lambda t, nb: (0, 0)),
            pl.BlockSpec((BN, DEGW), lambda t, nb: (nb, 0)),
        ],
        out_specs=pl.BlockSpec((1, BN, H), lambda t, nb: (t, nb, 0)),
        out_shape=jax.ShapeDtypeStruct((T, NPAD, H), jnp.float32),
    )(x_seq, W0, dis16)


def _mid_layer_body(p0_ref, p1_ref, hp_ref, dis_ref, b_ref, w_ref, out_ref):
    dis = dis_ref[:, 0:1]
    u = dis * (p0_ref[...] + p1_ref[...] + hp_ref[...]) + b_ref[...]
    u = jnp.maximum(u, 0.0)
    out_ref[...] = jnp.dot(u, w_ref[...], preferred_element_type=jnp.float32) * dis


def _mid_layer(P, hp, dis16, b, W_next):
    return pl.pallas_call(
        _mid_layer_body,
        grid=(NBLK,),
        in_specs=[
            pl.BlockSpec((BN, H), lambda nb: (nb, 0)),
            pl.BlockSpec((BN, H), lambda nb: (nb + NBLK, 0)),
            pl.BlockSpec((BN, H), lambda nb: (nb, 0)),
            pl.BlockSpec((BN, DEGW), lambda nb: (nb, 0)),
            pl.BlockSpec((1, H), lambda nb: (0, 0)),
            pl.BlockSpec((H, H), lambda nb: (0, 0)),
        ],
        out_specs=pl.BlockSpec((BN, H), lambda nb: (nb, 0)),
        out_shape=jax.ShapeDtypeStruct((NPAD, H), jnp.float32),
    )(P, P, hp, dis16, b, W_next)


def _last_layer_body(p0_ref, p1_ref, hp_ref, dis_ref, b_ref, bid_ref,
                     pooled_ref, counts_ref):
    nb = pl.program_id(0)
    dis = dis_ref[:, 0:1]
    u = dis * (p0_ref[...] + p1_ref[...] + hp_ref[...]) + b_ref[...]
    u = jnp.maximum(u, 0.0)
    rowid = nb * BN + lax.broadcasted_iota(jnp.int32, (BN, 1), 0)
    u = jnp.where(rowid < N, u, 0.0)
    bid = bid_ref[0, 0, :]
    iota_g = lax.broadcasted_iota(jnp.int32, (BN, G), 1)
    onehot = (bid[:, None] == iota_g).astype(jnp.float32)
    pooled_b = lax.dot_general(onehot, u, (((0,), (0,)), ((), ())),
                               preferred_element_type=jnp.float32)
    counts_b = jnp.sum(onehot, axis=0, keepdims=True)

    @pl.when(nb == 0)
    def _():
        pooled_ref[...] = pooled_b
        counts_ref[...] = counts_b

    @pl.when(nb > 0)
    def _():
        pooled_ref[...] += pooled_b
        counts_ref[...] += counts_b


def _last_layer(P, hp, dis16, b, bids3d):
    return pl.pallas_call(
        _last_layer_body,
        grid=(NBLK,),
        in_specs=[
            pl.BlockSpec((BN, H), lambda nb: (nb, 0)),
            pl.BlockSpec((BN, H), lambda nb: (nb + NBLK, 0)),
            pl.BlockSpec((BN, H), lambda nb: (nb, 0)),
            pl.BlockSpec((BN, DEGW), lambda nb: (nb, 0)),
            pl.BlockSpec((1, H), lambda nb: (0, 0)),
            pl.BlockSpec((1, 1, BN), lambda nb: (nb, 0, 0)),
        ],
        out_specs=[
            pl.BlockSpec((G, H), lambda nb: (0, 0)),
            pl.BlockSpec((1, G), lambda nb: (0, 0)),
        ],
        out_shape=[
            jax.ShapeDtypeStruct((G, H), jnp.float32),
            jax.ShapeDtypeStruct((1, G), jnp.float32),
        ],
    )(P, P, hp, dis16, b, bids3d)


def _sigmoid(x):
    return 1.0 / (1.0 + jnp.exp(-x))


def _gru_heads_body(*refs):
    pooled_refs = refs[:T]
    (counts_ref, wih_ref, whh_ref, bih_ref, bhh_ref,
     wc_ref, bc_ref, ws_ref, bs_ref, coord_ref, ss_ref) = refs[T:]
    inv = 1.0 / jnp.maximum(counts_ref[0, :], 1.0)
    h = jnp.zeros((G, H), jnp.float32)
    for t in range(T):
        xt = pooled_refs[t][...] * inv[:, None]
        gi = lax.dot_general(xt, wih_ref[...], (((1,), (1,)), ((), ())),
                             preferred_element_type=jnp.float32) + bih_ref[...]
        gh = lax.dot_general(h, whh_ref[...], (((1,), (1,)), ((), ())),
                             preferred_element_type=jnp.float32) + bhh_ref[...]
        r = _sigmoid(gi[:, 0:H] + gh[:, 0:H])
        z = _sigmoid(gi[:, H:2 * H] + gh[:, H:2 * H])
        n = jnp.tanh(gi[:, 2 * H:] + r * gh[:, 2 * H:])
        h = (1.0 - z) * n + z * h
        coord_ref[t] = jnp.dot(h, wc_ref[...],
                               preferred_element_type=jnp.float32) + bc_ref[...]
        ss_ref[t] = jnp.dot(h, ws_ref[...],
                            preferred_element_type=jnp.float32) + bs_ref[...]


def _gru_heads(pooled_list, counts, W_ih, W_hh, b_ih, b_hh,
               W_coord, b_coord, W_ss, b_ss):
    return pl.pallas_call(
        _gru_heads_body,
        out_shape=[
            jax.ShapeDtypeStruct((T, G, A * 3), jnp.float32),
            jax.ShapeDtypeStruct((T, G, A * S), jnp.float32),
        ],
    )(*pooled_list, counts, W_ih, W_hh, b_ih, b_hh,
      W_coord, b_coord, W_ss, b_ss)


def kernel(x_seq, edge_index, batch_ids, W0, b0, W1, b1, W2, b2, W3, b3,
           W_ih, W_hh, b_ih, b_hh, W_coord, b_coord, W_ss, b_ss):
    pad = jnp.zeros((NW, 2, CHUNK), jnp.int32)
    src2d = jnp.concatenate([edge_index[0].reshape(NW, NCHUNK, CHUNK), pad], 1)
    dst2d = jnp.concatenate([edge_index[1].reshape(NW, NCHUNK, CHUNK), pad], 1)
    bids_pad = jnp.full((NPAD,), G, jnp.int32).at[:N].set(batch_ids)
    bids3d = bids_pad.reshape(NBLK, 1, BN)
    b0r, b1r, b2r, b3r = (b.reshape(1, H) for b in (b0, b1, b2, b3))

    ones_mat = jnp.ones((NPAD, H), jnp.float32)
    degp = _agg_kernel(ones_mat, src2d, dst2d)
    dis16 = _dis_kernel(degp)

    hp0 = _first_layer(x_seq, W0, dis16)
    hp = [hp0[t] for t in range(T)]

    for (b, W_next) in [(b0r, W1), (b1r, W2), (b2r, W3)]:
        nxt = []
        for t in range(T):
            P = _agg_kernel(hp[t], src2d, dst2d)
            nxt.append(_mid_layer(P, hp[t], dis16, b, W_next))
        hp = nxt

    pooled_list = []
    counts = None
    for t in range(T):
        P = _agg_kernel(hp[t], src2d, dst2d)
        pooled_t, counts_t = _last_layer(P, hp[t], dis16, b3r, bids3d)
        pooled_list.append(pooled_t)
        if counts is None:
            counts = counts_t

    coord_r, ss_r = _gru_heads(
        pooled_list, counts, W_ih, W_hh, b_ih.reshape(1, 3 * H),
        b_hh.reshape(1, 3 * H), W_coord, b_coord.reshape(1, A * 3),
        W_ss, b_ss.reshape(1, A * S))

    coord = jnp.transpose(coord_r, (1, 0, 2)).reshape(G, T, A, 3)
    ss = jnp.transpose(ss_r, (1, 0, 2)).reshape(G, T, A, S)
    return (coord, ss)
